# Initial kernel scaffold; baseline (speedup 1.0000x reference)
import jax
import jax.numpy as jnp
from jax.experimental import pallas as pl


def kernel(bb_feats, segment_ids, W1, b1, W2, b2):
    raise NotImplementedError("write your pallas kernel here")


# speedup vs baseline: 6.4561x; 6.4561x over previous
import functools

import jax
import jax.numpy as jnp
from jax.experimental import pallas as pl
from jax.experimental.pallas import tpu as pltpu

N = 320000
D = 128
G = 256
B = 1280
NB = N // B


def _seg_mlp_kernel(ids_ref, x_ref, w1_ref, b1_ref, w2_ref, b2_ref,
                    out_ref, acc_ref, cnt_ref):
    i = pl.program_id(0)

    @pl.when(i == 0)
    def _init():
        acc_ref[...] = jnp.zeros_like(acc_ref)
        cnt_ref[...] = jnp.zeros_like(cnt_ref)

    ids = ids_ref[0, 0, :]
    x = x_ref[...]
    seg = jax.lax.broadcasted_iota(jnp.int32, (G, B), 0)
    onehot = (seg == ids[None, :]).astype(jnp.float32)
    acc_ref[...] += jnp.dot(onehot, x, preferred_element_type=jnp.float32)
    cnt_ref[...] += jnp.sum(onehot, axis=1, keepdims=False)[None, :]

    @pl.when(i == NB - 1)
    def _finish():
        counts = jnp.maximum(cnt_ref[0, :], 1.0)
        means = acc_ref[...] / counts[:, None]
        h = means @ w1_ref[...] + b1_ref[0, :][None, :]
        h = 0.5 * h * (1.0 + jax.lax.erf(h * 0.7071067811865476))
        z = h @ w2_ref[...] + b2_ref[0, :][None, :]
        out_ref[...] = jax.nn.softplus(z)


@jax.jit
def kernel(bb_feats, segment_ids, W1, b1, W2, b2):
    ids3 = segment_ids.astype(jnp.int32).reshape(NB, 1, B)
    W2p = jnp.zeros((D, 128), W2.dtype).at[:, :6].set(W2)
    b2p = jnp.zeros((1, 128), b2.dtype).at[0, :6].set(b2)
    b1p = b1.reshape(1, D)

    out = pl.pallas_call(
        _seg_mlp_kernel,
        grid=(NB,),
        in_specs=[
            pl.BlockSpec((1, 1, B), lambda i: (i, 0, 0)),
            pl.BlockSpec((B, D), lambda i: (i, 0)),
            pl.BlockSpec((D, D), lambda i: (0, 0)),
            pl.BlockSpec((1, D), lambda i: (0, 0)),
            pl.BlockSpec((D, 128), lambda i: (0, 0)),
            pl.BlockSpec((1, 128), lambda i: (0, 0)),
        ],
        out_specs=pl.BlockSpec((G, 128), lambda i: (0, 0)),
        out_shape=jax.ShapeDtypeStruct((G, 128), jnp.float32),
        scratch_shapes=[
            pltpu.VMEM((G, D), jnp.float32),
            pltpu.VMEM((1, G), jnp.float32),
        ],
    )(ids3, bb_feats, W1, b1p, W2p, b2p)
    return out[:, :6]
